# baseline (speedup 1.0000x reference)
import functools

import jax
import jax.numpy as jnp
from jax import lax
from jax.experimental import pallas as pl
from jax.experimental.pallas import tpu as pltpu
from jax.experimental.pallas import tpu_sc as plsc

N = 10000
E = 320000
D = 128

NC = 2
NS = 16
NW = NC * NS
CHUNK = 80
NCHUNKS = E // CHUNK
GSZ = 8
BLK = GSZ * CHUNK
NGROUPS = NCHUNKS // GSZ
NB = 3
ZCH = 80
NZCH = N // ZCH
TC_BLOCK = 1000

_mesh = plsc.VectorSubcoreMesh(core_axis_name="c", subcore_axis_name="s")


def _copy_idx(blk_ref, off, dst_ref):
    for i in range(CHUNK // 16):
        dst_ref[pl.ds(i * 16, 16)] = blk_ref[pl.ds(off + i * 16, 16)]


@functools.partial(
    pl.kernel,
    out_type=jax.ShapeDtypeStruct((2 * N,), jnp.float32),
    mesh=_mesh,
    scratch_types=[
        pltpu.VMEM((BLK,), jnp.int32),
        pltpu.VMEM((CHUNK,), jnp.int32),
        pltpu.VMEM((CHUNK,), jnp.float32),
        pltpu.VMEM((2000,), jnp.float32),
        pltpu.VMEM_SHARED((N,), jnp.float32),
    ],
)
def _deg_kernel(row_hbm, deg_hbm, blk_v, idx_v, ones_v, zeros_v, acc_sh):
    cid = lax.axis_index("c")
    sid = lax.axis_index("s")
    wid = sid * NC + cid

    one16 = jnp.ones((16,), jnp.float32)
    zero16 = jnp.zeros((16,), jnp.float32)
    for j in range(CHUNK // 16):
        ones_v[pl.ds(j * 16, 16)] = one16

    def fill_zero(i, _):
        zeros_v[pl.ds(i * 16, 16)] = zero16
        return 0

    lax.fori_loop(0, 2000 // 16, fill_zero, 0)

    @pl.when(sid < 5)
    def _():
        pltpu.sync_copy(zeros_v, acc_sh.at[pl.ds(sid * 2000, 2000)])

    plsc.subcore_barrier()

    n_g = (NGROUPS - wid + NW - 1) // NW

    def body(t, _):
        g = wid + NW * t
        pltpu.sync_copy(row_hbm.at[pl.ds(g * BLK, BLK)], blk_v)
        for j in range(GSZ):
            _copy_idx(blk_v, j * CHUNK, idx_v)
            pltpu.sync_copy(ones_v, acc_sh.at[idx_v], add=True)
        return 0

    lax.fori_loop(0, n_g, body, 0)

    plsc.subcore_barrier()

    @pl.when(sid < 10)
    def _():
        pltpu.sync_copy(acc_sh.at[pl.ds(sid * 1000, 1000)],
                        zeros_v.at[pl.ds(0, 1000)])
        pltpu.sync_copy(zeros_v.at[pl.ds(0, 1000)],
                        deg_hbm.at[pl.ds(cid * N + sid * 1000, 1000)])


def _linear_body(x_ref, w_ref, b_ref, d0_ref, d1_ref, h_ref, r_ref):
    deg = d0_ref[:, 0] + d1_ref[:, 0]
    r_full = lax.rsqrt(deg)
    r_safe = jnp.where(deg > 0.0, r_full, 0.0)
    m = lax.dot_general(
        x_ref[...], w_ref[...],
        dimension_numbers=(((1,), (1,)), ((), ())),
        preferred_element_type=jnp.float32,
    )
    h_ref[...] = r_full[:, None] * (m + b_ref[0, :][None, :])
    r_ref[...] = r_safe[:, None]


def _linear(x, W, b2, d0, d1):
    grid = (N // TC_BLOCK,)
    return pl.pallas_call(
        _linear_body,
        grid=grid,
        in_specs=[
            pl.BlockSpec((TC_BLOCK, D), lambda i: (i, 0)),
            pl.BlockSpec((D, D), lambda i: (0, 0)),
            pl.BlockSpec((1, D), lambda i: (0, 0)),
            pl.BlockSpec((TC_BLOCK, 1), lambda i: (i, 0)),
            pl.BlockSpec((TC_BLOCK, 1), lambda i: (i, 0)),
        ],
        out_specs=[
            pl.BlockSpec((TC_BLOCK, D), lambda i: (i, 0)),
            pl.BlockSpec((TC_BLOCK, 1), lambda i: (i, 0)),
        ],
        out_shape=[
            jax.ShapeDtypeStruct((N, D), jnp.float32),
            jax.ShapeDtypeStruct((N, 1), jnp.float32),
        ],
    )(x, W, b2, d0, d1)


@functools.partial(
    pl.kernel,
    out_type=jax.ShapeDtypeStruct((2 * N, D), jnp.float32),
    mesh=_mesh,
    scratch_types=[
        pltpu.VMEM((BLK,), jnp.int32),
        pltpu.VMEM((BLK,), jnp.int32),
        pltpu.VMEM((CHUNK,), jnp.int32),
        pltpu.VMEM((CHUNK,), jnp.int32),
        pltpu.VMEM((CHUNK,), jnp.int32),
        pltpu.VMEM((CHUNK,), jnp.int32),
        pltpu.VMEM((CHUNK,), jnp.int32),
        pltpu.VMEM((CHUNK, D), jnp.float32),
        pltpu.VMEM((CHUNK, D), jnp.float32),
        pltpu.VMEM((CHUNK, D), jnp.float32),
        pltpu.VMEM_SHARED((N, D), jnp.float32),
        pltpu.SemaphoreType.DMA,
        pltpu.SemaphoreType.DMA,
        pltpu.SemaphoreType.DMA,
        pltpu.SemaphoreType.DMA,
        pltpu.SemaphoreType.DMA,
    ],
)
def _agg_kernel(h_hbm, row_hbm, col_hbm, part_hbm,
                dst_blk, src_blk, dst0, dst1, src0, src1, src2,
                rows0, rows1, rows2, acc_sh,
                gsem0, gsem1, gsem2, ssem0, ssem1):
    cid = lax.axis_index("c")
    sid = lax.axis_index("s")
    wid = sid * NC + cid

    rows = (rows0, rows1, rows2)
    dsts = (dst0, dst1)
    srcs = (src0, src1, src2)
    gsems = (gsem0, gsem1, gsem2)
    ssems = (ssem0, ssem1)

    zero16 = jnp.zeros((16,), jnp.float32)

    def zrow(i, _):
        for j in range(D // 16):
            rows0[i, pl.ds(j * 16, 16)] = zero16
        return 0

    lax.fori_loop(0, ZCH, zrow, 0)

    n_z = (NZCH - sid + 15) // 16

    def zchunk(k, _):
        c = sid + 16 * k
        pltpu.sync_copy(rows0, acc_sh.at[pl.ds(c * ZCH, ZCH)])
        return 0

    lax.fori_loop(0, n_z, zchunk, 0)

    plsc.subcore_barrier()

    n_g = (NGROUPS - wid + NW - 1) // NW

    def body(t, _):
        g = wid + NW * t
        di = pltpu.async_copy(row_hbm.at[pl.ds(g * BLK, BLK)], dst_blk, gsem0)
        si = pltpu.async_copy(col_hbm.at[pl.ds(g * BLK, BLK)], src_blk, gsem1)
        di.wait()
        si.wait()

        gd = [None] * GSZ
        sd = [None] * GSZ
        sdone = set()

        def swait(k):
            if k >= 0 and k not in sdone:
                sd[k].wait()
                sdone.add(k)

        _copy_idx(src_blk, 0, srcs[0])
        gd[0] = pltpu.async_copy(h_hbm.at[srcs[0]], rows[0], gsems[0])
        _copy_idx(src_blk, CHUNK, srcs[1])
        gd[1] = pltpu.async_copy(h_hbm.at[srcs[1]], rows[1], gsems[1])

        for j in range(GSZ):
            if j + 2 < GSZ:
                swait(j - 1)
                _copy_idx(src_blk, (j + 2) * CHUNK, srcs[(j + 2) % NB])
                gd[j + 2] = pltpu.async_copy(
                    h_hbm.at[srcs[(j + 2) % NB]], rows[(j + 2) % NB],
                    gsems[(j + 2) % NB])
            gd[j].wait()
            swait(j - 2)
            _copy_idx(dst_blk, j * CHUNK, dsts[j % 2])
            sd[j] = pltpu.async_copy(
                rows[j % NB], acc_sh.at[dsts[j % 2]], ssems[j % 2], add=True)
        for k in range(GSZ):
            swait(k)
        return 0

    lax.fori_loop(0, n_g, body, 0)

    plsc.subcore_barrier()

    def wchunk(k, _):
        c = sid + 16 * k
        pltpu.sync_copy(acc_sh.at[pl.ds(c * ZCH, ZCH)], rows1)
        pltpu.sync_copy(rows1, part_hbm.at[pl.ds(cid * N + c * ZCH, ZCH)])
        return 0

    lax.fori_loop(0, n_z, wchunk, 0)


def _combine_body(p0_ref, p1_ref, r_ref, o_ref):
    o_ref[...] = (p0_ref[...] + p1_ref[...]) * r_ref[...]


def _combine(p0, p1, r):
    grid = (N // TC_BLOCK,)
    return pl.pallas_call(
        _combine_body,
        grid=grid,
        in_specs=[
            pl.BlockSpec((TC_BLOCK, D), lambda i: (i, 0)),
            pl.BlockSpec((TC_BLOCK, D), lambda i: (i, 0)),
            pl.BlockSpec((TC_BLOCK, 1), lambda i: (i, 0)),
        ],
        out_specs=pl.BlockSpec((TC_BLOCK, D), lambda i: (i, 0)),
        out_shape=jax.ShapeDtypeStruct((N, D), jnp.float32),
    )(p0, p1, r)


@jax.jit
def _impl(x, row, col, W, b):
    degp = _deg_kernel(row)
    d0 = degp[:N].reshape(N, 1)
    d1 = degp[N:].reshape(N, 1)
    h, r = _linear(x, W, b.reshape(1, D), d0, d1)
    parts = _agg_kernel(h, row, col)
    return _combine(parts[:N], parts[N:], r)


def kernel(x, edge_index, W, b):
    row = jnp.asarray(edge_index[0], jnp.int32)
    col = jnp.asarray(edge_index[1], jnp.int32)
    return _impl(x, row, col, W, b)
